# baseline (speedup 1.0000x reference)
import functools

import jax
import jax.numpy as jnp
from jax import lax
from jax.experimental import pallas as pl
from jax.experimental.pallas import tpu as pltpu
from jax.experimental.pallas import tpu_sc as plsc

_HID = 128
_C = 128
_NW = 32
_NBUF = 3
_BB = 16


def _sc_gather(tokens_flat, word_emb):
  N = tokens_flat.shape[0]
  per_w = N // _C // _NW
  npw = per_w * _C

  mesh = plsc.VectorSubcoreMesh(core_axis_name="c", subcore_axis_name="s")

  @functools.partial(
      pl.kernel,
      out_type=jax.ShapeDtypeStruct((N, _HID), jnp.float32),
      mesh=mesh,
      scratch_types=[
          pltpu.VMEM((npw,), jnp.int32),
          pltpu.VMEM((_NBUF, _C, _HID), jnp.float32),
          pltpu.SemaphoreType.DMA((_NBUF,)),
          pltpu.SemaphoreType.DMA((_NBUF,)),
      ],
  )
  def body(tok_hbm, word_hbm, out_hbm, tok_v, rows_v, in_sem, out_sem):
    wid = lax.axis_index("s") * 2 + lax.axis_index("c")
    w0 = wid * npw
    pltpu.sync_copy(tok_hbm.at[pl.ds(w0, npw)], tok_v)

    def issue(c):
      b = lax.rem(c, _NBUF)
      pltpu.async_copy(word_hbm.at[tok_v.at[pl.ds(c * _C, _C)]], rows_v.at[b],
                       in_sem.at[b])

    issue(0)
    issue(1)

    def loop_body(i, carry):
      b = lax.rem(i, _NBUF)
      pltpu.make_async_copy(word_hbm.at[pl.ds(0, _C)], rows_v.at[b],
                            in_sem.at[b]).wait()
      pltpu.async_copy(rows_v.at[b], out_hbm.at[pl.ds(w0 + i * _C, _C)],
                       out_sem.at[b])

      @pl.when(i + 2 < per_w)
      def _():
        b2 = lax.rem(i + 2, _NBUF)

        @pl.when(i >= 1)
        def _():
          pltpu.make_async_copy(rows_v.at[b2], out_hbm.at[pl.ds(0, _C)],
                                out_sem.at[b2]).wait()

        issue(i + 2)

      return carry

    lax.fori_loop(0, per_w, loop_body, 0)
    for k in range(_NBUF):
      pltpu.make_async_copy(rows_v.at[k], out_hbm.at[pl.ds(0, _C)],
                            out_sem.at[k]).wait()

  return body(tokens_flat, word_emb)


def _tc_ln_body(g_ref, s_ref, p_ref, t_ref, gm_ref, bt_ref, o_ref):
  x = g_ref[...]
  seg = s_ref[...]
  t0 = t_ref[0][None, None, :]
  t1 = t_ref[1][None, None, :]
  seg_b = lax.broadcast_in_dim(seg.astype(jnp.float32), x.shape, (0, 1))
  x = x + p_ref[...][None, :, :] + (t0 + seg_b * (t1 - t0))
  mean = jnp.mean(x, axis=-1, keepdims=True)
  var = jnp.mean(x * x, axis=-1, keepdims=True) - mean * mean
  y = lax.rsqrt(var + 1e-12)
  o_ref[...] = (x - mean) * y * gm_ref[...] + bt_ref[...]


def _tc_ln(gathered, segments, pos_emb_l, type_emb, gamma, beta):
  B, L = segments.shape
  g3 = gathered.reshape(B, L, _HID)
  grid = (B // _BB,)
  return pl.pallas_call(
      _tc_ln_body,
      grid=grid,
      in_specs=[
          pl.BlockSpec((_BB, L, _HID), lambda i: (i, 0, 0)),
          pl.BlockSpec((_BB, L), lambda i: (i, 0)),
          pl.BlockSpec((L, _HID), lambda i: (0, 0)),
          pl.BlockSpec((2, _HID), lambda i: (0, 0)),
          pl.BlockSpec((_HID,), lambda i: (0,)),
          pl.BlockSpec((_HID,), lambda i: (0,)),
      ],
      out_specs=pl.BlockSpec((_BB, L, _HID), lambda i: (i, 0, 0)),
      out_shape=jax.ShapeDtypeStruct((B, L, _HID), jnp.float32),
  )(g3, segments, pos_emb_l, type_emb, gamma, beta)


def kernel(tokens, segments, word_emb, pos_emb, type_emb, gamma, beta):
  B, L = tokens.shape
  tok_flat = tokens.astype(jnp.int32).reshape(-1)
  seg = segments.astype(jnp.int32)
  pos_l = pos_emb[:L]
  half = (B // 2) * L
  g0 = _sc_gather(tok_flat[:half], word_emb)
  g1 = _sc_gather(tok_flat[half:], word_emb)
  y0 = _tc_ln(g0, seg[: B // 2], pos_l, type_emb, gamma, beta)
  y1 = _tc_ln(g1, seg[B // 2 :], pos_l, type_emb, gamma, beta)
  return jnp.concatenate([y0, y1], axis=0)
